# baseline (speedup 1.0000x reference)
import functools
import math

import jax
import jax.numpy as jnp
import numpy as np
from jax import lax
from jax.experimental import pallas as pl
from jax.experimental.pallas import tpu as pltpu
from jax.experimental.pallas import tpu_sc as plsc

DIM = 1024
SCALE = math.sqrt(DIM)
LO = 512
SPLIT = 3584
SEQ = 4096
N_WORKERS = 32
SC_SEQ = SEQ - SPLIT
SC_PER_W = SC_SEQ // N_WORKERS


def _freqs():
    d = np.arange(DIM)
    return np.exp(-(2 * (d // 2)).astype(np.float64) * (math.log(10000.0) / DIM))


def _make_tc_tables(seq):
    freq = _freqs()
    even = (np.arange(DIM) % 2) == 0
    n_hi = seq // LO
    hi_angle = (LO * np.arange(n_hi, dtype=np.float64))[:, None] * freq[None, :]
    p_hi = np.where(even[None, :], np.sin(hi_angle), np.cos(hi_angle))
    q_hi = np.where(even[None, :], np.cos(hi_angle), -np.sin(hi_angle))
    lo_angle = np.arange(LO, dtype=np.float64)[:, None] * freq[None, :]
    return (
        p_hi.astype(np.float32)[:, None, :],
        q_hi.astype(np.float32)[:, None, :],
        np.cos(lo_angle).astype(np.float32),
        np.sin(lo_angle).astype(np.float32),
    )


def _make_pe(seq):
    freq = _freqs()
    even = (np.arange(DIM) % 2) == 0
    angle = np.arange(seq, dtype=np.float64)[:, None] * freq[None, :]
    return np.where(even[None, :], np.sin(angle), np.cos(angle)).astype(np.float32)


_TC_TABLES = _make_tc_tables(SPLIT)
_PE_SC = _make_pe(SEQ)[SPLIT:]


def _tc_block_kernel(emb_ref, p_ref, q_ref, cl_ref, sl_ref, out_ref):
    pe = p_ref[0] * cl_ref[...] + q_ref[0] * sl_ref[...]
    out_ref[...] = emb_ref[...] * SCALE + pe[:, None, :]


def _tc_part(emb):
    seq, feat, dim = emb.shape
    grid = (SPLIT // LO,)
    return pl.pallas_call(
        _tc_block_kernel,
        grid=grid,
        in_specs=[
            pl.BlockSpec((LO, feat, dim), lambda i: (i, 0, 0)),
            pl.BlockSpec((1, 1, dim), lambda i: (i, 0, 0)),
            pl.BlockSpec((1, 1, dim), lambda i: (i, 0, 0)),
            pl.BlockSpec((LO, dim), lambda i: (0, 0)),
            pl.BlockSpec((LO, dim), lambda i: (0, 0)),
        ],
        out_specs=pl.BlockSpec((LO, feat, dim), lambda i: (i, 0, 0)),
        out_shape=jax.ShapeDtypeStruct((SPLIT, feat, dim), emb.dtype),
    )(emb, *_TC_TABLES)


_SC_MESH = plsc.VectorSubcoreMesh(
    core_axis_name="c", subcore_axis_name="s", num_cores=2, num_subcores=16
)


@functools.partial(
    pl.kernel,
    out_type=jax.ShapeDtypeStruct((SC_SEQ, 4, DIM), jnp.float32),
    mesh=_SC_MESH,
    scratch_types=[
        pltpu.VMEM((SC_PER_W, 4, DIM), jnp.float32),
        pltpu.VMEM((SC_PER_W, DIM), jnp.float32),
    ],
)
def _sc_part(emb_hbm, pe_hbm, out_hbm, emb_v, pe_v):
    wid = lax.axis_index("s") * 2 + lax.axis_index("c")
    base = wid * SC_PER_W
    pltpu.sync_copy(emb_hbm.at[pl.ds(SPLIT + base, SC_PER_W)], emb_v)
    pltpu.sync_copy(pe_hbm.at[pl.ds(base, SC_PER_W)], pe_v)

    for s in range(SC_PER_W):
        def body(g, carry, s=s):
            sl = pl.ds(g * 16, 16)
            p = pe_v[s, sl]
            for f in range(4):
                emb_v[s, f, sl] = emb_v[s, f, sl] * SCALE + p
            return carry
        lax.fori_loop(0, DIM // 16, body, 0)

    pltpu.sync_copy(emb_v, out_hbm.at[pl.ds(base, SC_PER_W)])


def kernel(emb):
    tc_out = _tc_part(emb)
    sc_out = _sc_part(emb, jnp.asarray(_PE_SC))
    return jnp.concatenate([tc_out, sc_out], axis=0)


# speedup vs baseline: 2.9871x; 2.9871x over previous
import math

import jax
import jax.numpy as jnp
import numpy as np
from jax.experimental import pallas as pl

DIM = 1024
SCALE = math.sqrt(DIM)
LO = 512


def _make_tables(seq):
    d = np.arange(DIM)
    freq = np.exp(-(2 * (d // 2)).astype(np.float64) * (math.log(10000.0) / DIM))
    even = (d % 2) == 0

    n_hi = seq // LO
    hi_angle = (LO * np.arange(n_hi, dtype=np.float64))[:, None] * freq[None, :]
    p_hi = np.where(even[None, :], np.sin(hi_angle), np.cos(hi_angle))
    q_hi = np.where(even[None, :], np.cos(hi_angle), -np.sin(hi_angle))

    lo_angle = np.arange(LO, dtype=np.float64)[:, None] * freq[None, :]
    c_lo = np.cos(lo_angle)
    s_lo = np.sin(lo_angle)

    return (
        p_hi.astype(np.float32)[:, None, :],
        q_hi.astype(np.float32)[:, None, :],
        c_lo.astype(np.float32),
        s_lo.astype(np.float32),
    )


_TABLES = _make_tables(4096)


def _block_kernel(emb_ref, p_ref, q_ref, cl_ref, sl_ref, out_ref):
    i = pl.program_id(0)
    pe = p_ref[i] * cl_ref[...] + q_ref[i] * sl_ref[...]
    out_ref[...] = emb_ref[...] * SCALE + pe[:, None, :]


def kernel(emb):
    seq, feat, dim = emb.shape
    n_hi = seq // LO
    grid = (n_hi,)
    return pl.pallas_call(
        _block_kernel,
        grid=grid,
        in_specs=[
            pl.BlockSpec((LO, feat, dim), lambda i: (i, 0, 0)),
            pl.BlockSpec((n_hi, 1, dim), lambda i: (0, 0, 0)),
            pl.BlockSpec((n_hi, 1, dim), lambda i: (0, 0, 0)),
            pl.BlockSpec((LO, dim), lambda i: (0, 0)),
            pl.BlockSpec((LO, dim), lambda i: (0, 0)),
        ],
        out_specs=pl.BlockSpec((LO, feat, dim), lambda i: (i, 0, 0)),
        out_shape=jax.ShapeDtypeStruct((seq, feat, dim), emb.dtype),
    )(emb, *_TABLES)


# speedup vs baseline: 3.0598x; 1.0243x over previous
import math

import jax
import jax.numpy as jnp
import numpy as np
from jax.experimental import pallas as pl

DIM = 1024
SCALE = math.sqrt(DIM)
LO = 512
K = 8
LSUB = LO // K


def _make_tables(seq):
    d = np.arange(DIM)
    freq = np.exp(-(2 * (d // 2)).astype(np.float64) * (math.log(10000.0) / DIM))
    even = (d % 2) == 0

    n_hi = seq // LSUB
    hi_angle = (LSUB * np.arange(n_hi, dtype=np.float64))[:, None] * freq[None, :]
    p_hi = np.where(even[None, :], np.sin(hi_angle), np.cos(hi_angle))
    q_hi = np.where(even[None, :], np.cos(hi_angle), -np.sin(hi_angle))

    lo_angle = np.arange(LSUB, dtype=np.float64)[:, None] * freq[None, :]
    c_lo = np.cos(lo_angle)
    s_lo = np.sin(lo_angle)

    return (
        p_hi.astype(np.float32)[:, None, :],
        q_hi.astype(np.float32)[:, None, :],
        c_lo.astype(np.float32),
        s_lo.astype(np.float32),
    )


_TABLES = _make_tables(4096)


def _block_kernel(emb_ref, p_ref, q_ref, cl_ref, sl_ref, out_ref):
    i = pl.program_id(0)
    cl = cl_ref[...]
    sl = sl_ref[...]
    pe = jnp.concatenate(
        [p_ref[i * K + k] * cl + q_ref[i * K + k] * sl for k in range(K)],
        axis=0,
    )
    out_ref[...] = emb_ref[...] * SCALE + pe[:, None, :]


def kernel(emb):
    seq, feat, dim = emb.shape
    n_hi = seq // LSUB
    grid = (seq // LO,)
    return pl.pallas_call(
        _block_kernel,
        grid=grid,
        in_specs=[
            pl.BlockSpec((LO, feat, dim), lambda i: (i, 0, 0)),
            pl.BlockSpec((n_hi, 1, dim), lambda i: (0, 0, 0)),
            pl.BlockSpec((n_hi, 1, dim), lambda i: (0, 0, 0)),
            pl.BlockSpec((LSUB, dim), lambda i: (0, 0)),
            pl.BlockSpec((LSUB, dim), lambda i: (0, 0)),
        ],
        out_specs=pl.BlockSpec((LO, feat, dim), lambda i: (i, 0, 0)),
        out_shape=jax.ShapeDtypeStruct((seq, feat, dim), emb.dtype),
    )(emb, *_TABLES)
